# Initial kernel scaffold; baseline (speedup 1.0000x reference)
import jax
import jax.numpy as jnp
from jax import lax
from jax.experimental import pallas as pl
from jax.experimental.pallas import tpu as pltpu
from jax.experimental.pallas import tpu_sc as plsc

NUM_PAGES = 32768
TOKENS_PER_PAGE = 64
SLOTS = 64
MAX_PAGES_PER_SLOT = 512

NC = 2
NS = 16
L = 16
CHUNK = NUM_PAGES // NS
NVREG = CHUNK // L
PM_ROWS = SLOTS // NS
SLOT_VREGS = SLOTS // L

_mesh = plsc.VectorSubcoreMesh(
    core_axis_name="c", subcore_axis_name="s", num_cores=NC, num_subcores=NS)

_out_type = (
    jax.ShapeDtypeStruct((NUM_PAGES,), jnp.int32),
    jax.ShapeDtypeStruct((SLOTS, MAX_PAGES_PER_SLOT), jnp.int32),
    jax.ShapeDtypeStruct((SLOTS,), jnp.int32),
    jax.ShapeDtypeStruct((SLOTS,), jnp.int32),
    jax.ShapeDtypeStruct((SLOTS,), jnp.int32),
    jax.ShapeDtypeStruct((SLOTS,), jnp.int32),
)

_scratch = [
    pltpu.VMEM((CHUNK,), jnp.int32),
    pltpu.VMEM((PM_ROWS, MAX_PAGES_PER_SLOT), jnp.int32),
    pltpu.VMEM((SLOTS, L), jnp.int32),
    pltpu.VMEM((SLOTS,), jnp.int32),
    pltpu.VMEM((L,), jnp.int32),
    pltpu.VMEM((NS, L), jnp.int32),
    pltpu.VMEM((SLOTS + 1, L), jnp.int32),
    pltpu.VMEM((SLOTS,), jnp.int32),
    pltpu.VMEM((SLOTS,), jnp.int32),
    pltpu.VMEM((SLOTS,), jnp.int32),
    pltpu.VMEM((SLOTS,), jnp.int32),
    pltpu.VMEM((SLOTS,), jnp.int32),
    pltpu.VMEM((SLOTS,), jnp.int32),
    pltpu.VMEM((SLOTS,), jnp.int32),
    pltpu.VMEM((SLOTS,), jnp.int32),
    pltpu.VMEM((SLOTS,), jnp.int32),
    pltpu.VMEM_SHARED((NS, L), jnp.int32),
    pltpu.VMEM_SHARED((SLOTS + 1, L), jnp.int32),
]


def _body(ps_hbm, pm_hbm, seq_hbm, npu_hbm, cp_hbm, cpp_hbm,
          ps_out, pm_out, nseq_out, nnpu_out, ncp_out, ncpp_out,
          chunk_v, pmbuf_v, src_v, idx_v, cntpub_v, cnts_v, cstage_v,
          seqb_v, npub_v, cpb_v, cslot_v, nslot_v,
          oseq_v, onpu_v, ocp_v, ocpp_v,
          shared_cnt, shared_claim):
    del cpp_hbm
    c = lax.axis_index("c")
    s = lax.axis_index("s")
    lanes = lax.iota(jnp.int32, L)
    zero16 = jnp.zeros((L,), jnp.int32)
    one16 = jnp.full((L,), 1, jnp.int32)
    base = s * CHUNK

    pltpu.sync_copy(ps_hbm.at[pl.ds(base, CHUNK)], chunk_v)
    pltpu.sync_copy(seq_hbm, seqb_v)
    pltpu.sync_copy(npu_hbm, npub_v)
    pltpu.sync_copy(cp_hbm, cpb_v)

    @pl.when(c == 1)
    def _():
        pltpu.sync_copy(pm_hbm.at[pl.ds(s * PM_ROWS, PM_ROWS)], pmbuf_v)

    @pl.when(s == 0)
    def _():
        for j in range(SLOTS + 1):
            cstage_v[j, :] = zero16
        pltpu.sync_copy(cstage_v, shared_claim)

    def _cond(carry):
        i, cnt = carry
        return jnp.logical_and(i < NVREG, cnt < SLOTS)

    def _scan(carry):
        i, cnt = carry
        v = chunk_v[pl.ds(i * L, L)]
        gidx = base + i * L + lanes
        m = jnp.logical_and(v == 0, gidx >= 1)
        m32 = jnp.where(m, 1, 0).astype(jnp.int32)
        csum = plsc.cumsum(m32)
        rank = cnt + csum - 1
        ok = jnp.logical_and(m, rank < SLOTS)
        plsc.store_scatter(
            src_v, [jnp.clip(rank, 0, SLOTS - 1), zero16], gidx, mask=ok)
        return i + 1, cnt + jnp.sum(m32)

    _, cnt = lax.while_loop(_cond, _scan, (jnp.int32(0), jnp.int32(0)))

    cntpub_v[...] = jnp.broadcast_to(cnt, (L,))
    pltpu.sync_copy(cntpub_v, shared_cnt.at[s])

    plsc.subcore_barrier()

    pltpu.sync_copy(shared_cnt, cnts_v)
    cnts16 = plsc.load_gather(cnts_v, [lanes, zero16])
    r_self = jnp.sum(jnp.where(lanes < s, cnts16, 0).astype(jnp.int32))
    total = jnp.sum(cnts16)
    for k in range(SLOT_VREGS):
        j = k * L + lanes
        g = jnp.minimum(r_self + j, jnp.int32(SLOTS))
        gi = jnp.where(j < cnt, g, jnp.int32(SLOTS))
        idx_v[pl.ds(k * L, L)] = gi
    pltpu.sync_copy(src_v, shared_claim.at[idx_v], add=True)

    plsc.subcore_barrier()

    pltpu.sync_copy(shared_claim, cstage_v)
    rank0 = jnp.int32(0)
    claims = []
    for k in range(SLOT_VREGS):
        off = k * L
        sq = seqb_v[pl.ds(off, L)]
        np_ = npub_v[pl.ds(off, L)]
        cp_ = cpb_v[pl.ds(off, L)]
        nsq = sq + jnp.where(cp_ == -1, 0, 1).astype(jnp.int32)
        nnp = (nsq + (TOKENS_PER_PAGE - 1)) // TOKENS_PER_PAGE
        ncpp = jnp.where(nsq == 0, 0, (nsq - 1) % TOKENS_PER_PAGE)
        needs = nnp > np_
        n32 = jnp.where(needs, 1, 0).astype(jnp.int32)
        csg = plsc.cumsum(n32)
        grank = rank0 + csg - 1
        rank0 = rank0 + jnp.sum(n32)
        valid = jnp.logical_and(needs, grank < total)
        rclip = jnp.clip(grank, 0, SLOTS - 1)
        page = plsc.load_gather(cstage_v, [rclip, zero16], mask=valid)
        page = jnp.where(valid, page, 0)
        ncp = jnp.where(needs, page, cp_)
        oseq_v[pl.ds(off, L)] = nsq
        onpu_v[pl.ds(off, L)] = nnp
        ocp_v[pl.ds(off, L)] = ncp
        ocpp_v[pl.ds(off, L)] = ncpp
        cslot_v[pl.ds(off, L)] = page
        nslot_v[pl.ds(off, L)] = n32
        claims.append(jnp.where(needs, page, -1))

    @pl.when(c == 0)
    def _():
        for k in range(SLOT_VREGS):
            p = claims[k]
            m = jnp.logical_and(p >= base, p < base + CHUNK)
            local = jnp.clip(p - base, 0, CHUNK - 1)
            plsc.store_scatter(chunk_v, [local], one16, mask=m)
        pltpu.sync_copy(chunk_v, ps_out.at[pl.ds(base, CHUNK)])

    @pl.when(jnp.logical_and(c == 0, s == 0))
    def _():
        pltpu.sync_copy(oseq_v, nseq_out)
        pltpu.sync_copy(onpu_v, nnpu_out)
        pltpu.sync_copy(ocp_v, ncp_out)
        pltpu.sync_copy(ocpp_v, ncpp_out)

    @pl.when(c == 1)
    def _():
        row4 = jnp.clip(s * PM_ROWS + lanes, 0, SLOTS - 1)
        m4 = lanes < PM_ROWS
        need4 = plsc.load_gather(nslot_v, [row4], mask=m4)
        page4 = plsc.load_gather(cslot_v, [row4], mask=m4)
        npu4 = plsc.load_gather(npub_v, [row4], mask=m4)
        wm = jnp.logical_and(m4, need4 == 1)
        plsc.store_scatter(
            pmbuf_v,
            [jnp.minimum(lanes, PM_ROWS - 1),
             jnp.clip(npu4, 0, MAX_PAGES_PER_SLOT - 1)],
            page4, mask=wm)
        pltpu.sync_copy(pmbuf_v, pm_out.at[pl.ds(s * PM_ROWS, PM_ROWS)])


_paged = pl.kernel(
    _body, out_type=_out_type, mesh=_mesh, scratch_types=_scratch,
    name="page_manager_sc")


def kernel(page_status, page_map, sequence_lengths, num_pages_used,
           current_page, current_page_position):
    return _paged(page_status, page_map, sequence_lengths, num_pages_used,
                  current_page, current_page_position)


# speedup vs baseline: 7.7517x; 7.7517x over previous
import jax
import jax.numpy as jnp
from jax import lax
from jax.experimental import pallas as pl
from jax.experimental.pallas import tpu as pltpu
from jax.experimental.pallas import tpu_sc as plsc

NUM_PAGES = 32768
TOKENS_PER_PAGE = 64
SLOTS = 64
MAX_PAGES_PER_SLOT = 512

NC = 2
NS = 16
L = 16
CHUNK = NUM_PAGES // NS
NVREG = CHUNK // L
PM_ROWS = SLOTS // NS
SLOT_VREGS = SLOTS // L

_mesh = plsc.VectorSubcoreMesh(
    core_axis_name="c", subcore_axis_name="s", num_cores=NC, num_subcores=NS)

_out_type = (
    jax.ShapeDtypeStruct((NUM_PAGES,), jnp.int32),
    jax.ShapeDtypeStruct((SLOTS, MAX_PAGES_PER_SLOT), jnp.int32),
    jax.ShapeDtypeStruct((SLOTS,), jnp.int32),
    jax.ShapeDtypeStruct((SLOTS,), jnp.int32),
    jax.ShapeDtypeStruct((SLOTS,), jnp.int32),
    jax.ShapeDtypeStruct((SLOTS,), jnp.int32),
)

_scratch = [
    pltpu.VMEM((CHUNK,), jnp.int32),
    pltpu.VMEM((PM_ROWS, MAX_PAGES_PER_SLOT), jnp.int32),
    pltpu.VMEM((SLOTS, L), jnp.int32),
    pltpu.VMEM((SLOTS,), jnp.int32),
    pltpu.VMEM((L,), jnp.int32),
    pltpu.VMEM((NS, L), jnp.int32),
    pltpu.VMEM((SLOTS + 1, L), jnp.int32),
    pltpu.VMEM((SLOTS,), jnp.int32),
    pltpu.VMEM((SLOTS,), jnp.int32),
    pltpu.VMEM((SLOTS,), jnp.int32),
    pltpu.VMEM((SLOTS,), jnp.int32),
    pltpu.VMEM((SLOTS,), jnp.int32),
    pltpu.VMEM((SLOTS,), jnp.int32),
    pltpu.VMEM((SLOTS,), jnp.int32),
    pltpu.VMEM((SLOTS,), jnp.int32),
    pltpu.VMEM((SLOTS,), jnp.int32),
    pltpu.VMEM_SHARED((NS, L), jnp.int32),
    pltpu.VMEM_SHARED((SLOTS + 1, L), jnp.int32),
]


def _body(ps_hbm, pm_hbm, seq_hbm, npu_hbm, cp_hbm, cpp_hbm,
          ps_out, pm_out, nseq_out, nnpu_out, ncp_out, ncpp_out,
          chunk_v, pmbuf_v, src_v, idx_v, cntpub_v, cnts_v, cstage_v,
          seqb_v, npub_v, cpb_v, cslot_v, nslot_v,
          oseq_v, onpu_v, ocp_v, ocpp_v,
          shared_cnt, shared_claim):
    del cpp_hbm
    c = lax.axis_index("c")
    s = lax.axis_index("s")
    lanes = lax.iota(jnp.int32, L)
    zero16 = jnp.zeros((L,), jnp.int32)
    one16 = jnp.full((L,), 1, jnp.int32)
    base = s * CHUNK

    pltpu.sync_copy(ps_hbm.at[pl.ds(base, CHUNK)], chunk_v)
    pltpu.sync_copy(seq_hbm, seqb_v)
    pltpu.sync_copy(npu_hbm, npub_v)
    pltpu.sync_copy(cp_hbm, cpb_v)

    @pl.when(c == 1)
    def _():
        pltpu.sync_copy(pm_hbm.at[pl.ds(s * PM_ROWS, PM_ROWS)], pmbuf_v)

    @pl.when(s == 0)
    def _():
        for j in range(SLOTS + 1):
            cstage_v[j, :] = zero16
        pltpu.sync_copy(cstage_v, shared_claim)

    def _scan(i, cnt):
        v = chunk_v[pl.ds(i * L, L)]
        gidx = base + i * L + lanes
        m = jnp.logical_and(v == 0, gidx >= 1)
        m32 = jnp.where(m, 1, 0).astype(jnp.int32)
        csum = plsc.cumsum(m32)
        rank = cnt + csum - 1
        ok = jnp.logical_and(m, rank < SLOTS)
        plsc.store_scatter(
            src_v, [jnp.clip(rank, 0, SLOTS - 1), zero16], gidx, mask=ok)
        return cnt + jnp.sum(m32)

    cnt = lax.fori_loop(0, NVREG, _scan, jnp.int32(0))

    cntpub_v[...] = jnp.broadcast_to(cnt, (L,))
    pltpu.sync_copy(cntpub_v, shared_cnt.at[s])

    plsc.subcore_barrier()

    pltpu.sync_copy(shared_cnt, cnts_v)
    cnts16 = plsc.load_gather(cnts_v, [lanes, zero16])
    r_self = jnp.sum(jnp.where(lanes < s, cnts16, 0).astype(jnp.int32))
    total = jnp.sum(cnts16)
    for k in range(SLOT_VREGS):
        j = k * L + lanes
        g = jnp.minimum(r_self + j, jnp.int32(SLOTS))
        gi = jnp.where(j < cnt, g, jnp.int32(SLOTS))
        idx_v[pl.ds(k * L, L)] = gi
    pltpu.sync_copy(src_v, shared_claim.at[idx_v], add=True)

    plsc.subcore_barrier()

    pltpu.sync_copy(shared_claim, cstage_v)
    rank0 = jnp.int32(0)
    claims = []
    for k in range(SLOT_VREGS):
        off = k * L
        sq = seqb_v[pl.ds(off, L)]
        np_ = npub_v[pl.ds(off, L)]
        cp_ = cpb_v[pl.ds(off, L)]
        nsq = sq + jnp.where(cp_ == -1, 0, 1).astype(jnp.int32)
        nnp = (nsq + (TOKENS_PER_PAGE - 1)) // TOKENS_PER_PAGE
        ncpp = jnp.where(nsq == 0, 0, (nsq - 1) % TOKENS_PER_PAGE)
        needs = nnp > np_
        n32 = jnp.where(needs, 1, 0).astype(jnp.int32)
        csg = plsc.cumsum(n32)
        grank = rank0 + csg - 1
        rank0 = rank0 + jnp.sum(n32)
        valid = jnp.logical_and(needs, grank < total)
        rclip = jnp.clip(grank, 0, SLOTS - 1)
        page = plsc.load_gather(cstage_v, [rclip, zero16], mask=valid)
        page = jnp.where(valid, page, 0)
        ncp = jnp.where(needs, page, cp_)
        oseq_v[pl.ds(off, L)] = nsq
        onpu_v[pl.ds(off, L)] = nnp
        ocp_v[pl.ds(off, L)] = ncp
        ocpp_v[pl.ds(off, L)] = ncpp
        cslot_v[pl.ds(off, L)] = page
        nslot_v[pl.ds(off, L)] = n32
        claims.append(jnp.where(needs, page, -1))

    @pl.when(c == 0)
    def _():
        for k in range(SLOT_VREGS):
            p = claims[k]
            m = jnp.logical_and(p >= base, p < base + CHUNK)
            local = jnp.clip(p - base, 0, CHUNK - 1)
            plsc.store_scatter(chunk_v, [local], one16, mask=m)
        pltpu.sync_copy(chunk_v, ps_out.at[pl.ds(base, CHUNK)])

    @pl.when(jnp.logical_and(c == 0, s == 0))
    def _():
        pltpu.sync_copy(oseq_v, nseq_out)
        pltpu.sync_copy(onpu_v, nnpu_out)
        pltpu.sync_copy(ocp_v, ncp_out)
        pltpu.sync_copy(ocpp_v, ncpp_out)

    @pl.when(c == 1)
    def _():
        row4 = jnp.clip(s * PM_ROWS + lanes, 0, SLOTS - 1)
        m4 = lanes < PM_ROWS
        need4 = plsc.load_gather(nslot_v, [row4], mask=m4)
        page4 = plsc.load_gather(cslot_v, [row4], mask=m4)
        npu4 = plsc.load_gather(npub_v, [row4], mask=m4)
        wm = jnp.logical_and(m4, need4 == 1)
        plsc.store_scatter(
            pmbuf_v,
            [jnp.minimum(lanes, PM_ROWS - 1),
             jnp.clip(npu4, 0, MAX_PAGES_PER_SLOT - 1)],
            page4, mask=wm)
        pltpu.sync_copy(pmbuf_v, pm_out.at[pl.ds(s * PM_ROWS, PM_ROWS)])


_paged = pl.kernel(
    _body, out_type=_out_type, mesh=_mesh, scratch_types=_scratch,
    compiler_params=pltpu.CompilerParams(needs_layout_passes=False),
    name="page_manager_sc")


def kernel(page_status, page_map, sequence_lengths, num_pages_used,
           current_page, current_page_position):
    return _paged(page_status, page_map, sequence_lengths, num_pages_used,
                  current_page, current_page_position)


# speedup vs baseline: 8.8903x; 1.1469x over previous
import jax
import jax.numpy as jnp
from jax import lax
from jax.experimental import pallas as pl
from jax.experimental.pallas import tpu as pltpu
from jax.experimental.pallas import tpu_sc as plsc

NUM_PAGES = 32768
TOKENS_PER_PAGE = 64
SLOTS = 64
MAX_PAGES_PER_SLOT = 512

NC = 2
NS = 16
L = 16
CHUNK = NUM_PAGES // NS
NVREG = CHUNK // L
PM_ROWS = SLOTS // NS
SLOT_VREGS = SLOTS // L
SB_VREGS = 16
NSB = NVREG // SB_VREGS
PROW = SLOTS + L

_mesh = plsc.VectorSubcoreMesh(
    core_axis_name="c", subcore_axis_name="s", num_cores=NC, num_subcores=NS)

_out_type = (
    jax.ShapeDtypeStruct((NUM_PAGES,), jnp.int32),
    jax.ShapeDtypeStruct((SLOTS, MAX_PAGES_PER_SLOT), jnp.int32),
    jax.ShapeDtypeStruct((SLOTS,), jnp.int32),
    jax.ShapeDtypeStruct((SLOTS,), jnp.int32),
    jax.ShapeDtypeStruct((SLOTS,), jnp.int32),
    jax.ShapeDtypeStruct((SLOTS,), jnp.int32),
)

_scratch = [
    pltpu.VMEM((CHUNK,), jnp.int32),
    pltpu.VMEM((PM_ROWS, MAX_PAGES_PER_SLOT), jnp.int32),
    pltpu.VMEM((PROW,), jnp.int32),
    pltpu.VMEM((NS, PROW), jnp.int32),
    pltpu.VMEM((L,), jnp.int32),
    pltpu.VMEM((L,), jnp.int32),
    pltpu.VMEM((SLOTS,), jnp.int32),
    pltpu.VMEM((SLOTS,), jnp.int32),
    pltpu.VMEM((SLOTS,), jnp.int32),
    pltpu.VMEM((SLOTS,), jnp.int32),
    pltpu.VMEM((SLOTS,), jnp.int32),
    pltpu.VMEM((SLOTS,), jnp.int32),
    pltpu.VMEM((SLOTS,), jnp.int32),
    pltpu.VMEM((SLOTS,), jnp.int32),
    pltpu.VMEM((SLOTS,), jnp.int32),
    pltpu.VMEM((SLOTS,), jnp.int32),
    pltpu.SMEM((1,), jnp.int32),
    pltpu.SemaphoreType.DMA,
    pltpu.VMEM_SHARED((NS, PROW), jnp.int32),
]


def _body(ps_hbm, pm_hbm, seq_hbm, npu_hbm, cp_hbm, cpp_hbm,
          ps_out, pm_out, nseq_out, nnpu_out, ncp_out, ncpp_out,
          chunk_v, pmbuf_v, pos_v, posall_v, rexcl_v, rincl_v, claim_v,
          seqb_v, npub_v, cpb_v, cslot_v, nslot_v,
          oseq_v, onpu_v, ocp_v, ocpp_v,
          cnt_smem, dma_sem, shared_pos):
    del cpp_hbm
    c = lax.axis_index("c")
    s = lax.axis_index("s")
    lanes = lax.iota(jnp.int32, L)
    zero16 = jnp.zeros((L,), jnp.int32)
    one16 = jnp.full((L,), 1, jnp.int32)
    base = s * CHUNK

    cps = pltpu.async_copy(ps_hbm.at[pl.ds(base, CHUNK)], chunk_v, dma_sem)
    csq = pltpu.async_copy(seq_hbm, seqb_v, dma_sem)
    cnp = pltpu.async_copy(npu_hbm, npub_v, dma_sem)
    ccp = pltpu.async_copy(cp_hbm, cpb_v, dma_sem)
    cpm = pltpu.async_copy(
        pm_hbm.at[pl.ds(s * PM_ROWS, PM_ROWS)], pmbuf_v, dma_sem)
    cps.wait()
    csq.wait()
    cnp.wait()
    ccp.wait()
    cpm.wait()

    cnt_smem[0] = jnp.int32(0)

    def _sb(b, carry):
        cnt0 = cnt_smem[0]

        @pl.when(cnt0 < SLOTS)
        def _():
            cnt = cnt0
            for u in range(SB_VREGS):
                off = b * (SB_VREGS * L) + u * L
                v = chunk_v[pl.ds(off, L)]
                gidx = base + off + lanes
                m = jnp.logical_and(v == 0, gidx >= 1)
                pc = plsc.all_reduce_population_count(m)
                plsc.store_compressed(
                    pos_v.at[pl.ds(jnp.minimum(cnt, SLOTS), L)], gidx, mask=m)
                cnt = cnt + pc[0]
            cnt_smem[0] = cnt

        return carry

    lax.fori_loop(0, NSB, _sb, jnp.int32(0))
    cnt = cnt_smem[0]

    pos_v[pl.ds(SLOTS, L)] = jnp.broadcast_to(cnt, (L,))
    pltpu.sync_copy(pos_v, shared_pos.at[s])

    plsc.subcore_barrier()

    pltpu.sync_copy(shared_pos, posall_v)
    cnts16 = plsc.load_gather(posall_v, [lanes, jnp.full((L,), SLOTS, jnp.int32)])
    rincl = plsc.cumsum(cnts16)
    rexcl_v[...] = rincl - cnts16
    rincl_v[...] = rincl
    total16 = plsc.load_gather(rincl_v, [jnp.full((L,), NS - 1, jnp.int32)])

    for kg in range(SLOT_VREGS):
        rg = kg * L + lanes
        tsel = zero16
        rsel = zero16
        for t in range(NS):
            rt = plsc.load_gather(rexcl_v, [jnp.full((L,), t, jnp.int32)])
            cond = rt <= rg
            tsel = jnp.where(cond, jnp.full((L,), t, jnp.int32), tsel)
            rsel = jnp.where(cond, rt, rsel)
        local = jnp.clip(rg - rsel, 0, SLOTS - 1)
        validg = rg < total16
        pg = plsc.load_gather(posall_v, [tsel, local], mask=validg)
        claim_v[pl.ds(kg * L, L)] = jnp.where(validg, pg, 0)

    rank0 = jnp.int32(0)
    claims = []
    for k in range(SLOT_VREGS):
        off = k * L
        sq = seqb_v[pl.ds(off, L)]
        np_ = npub_v[pl.ds(off, L)]
        cp_ = cpb_v[pl.ds(off, L)]
        nsq = sq + jnp.where(cp_ == -1, 0, 1).astype(jnp.int32)
        nnp = (nsq + (TOKENS_PER_PAGE - 1)) // TOKENS_PER_PAGE
        ncpp = jnp.where(nsq == 0, 0, (nsq - 1) % TOKENS_PER_PAGE)
        needs = nnp > np_
        n32 = jnp.where(needs, 1, 0).astype(jnp.int32)
        csg = plsc.cumsum(n32)
        grank = rank0 + csg - 1
        rank0 = rank0 + jnp.sum(n32)
        rclip = jnp.clip(grank, 0, SLOTS - 1)
        page = plsc.load_gather(claim_v, [rclip], mask=needs)
        page = jnp.where(needs, page, 0)
        ncp = jnp.where(needs, page, cp_)
        oseq_v[pl.ds(off, L)] = nsq
        onpu_v[pl.ds(off, L)] = nnp
        ocp_v[pl.ds(off, L)] = ncp
        ocpp_v[pl.ds(off, L)] = ncpp
        cslot_v[pl.ds(off, L)] = page
        nslot_v[pl.ds(off, L)] = n32
        claims.append(jnp.where(needs, page, -1))

    @pl.when(c == 0)
    def _():
        for k in range(SLOT_VREGS):
            p = claims[k]
            m = jnp.logical_and(p >= base, p < base + CHUNK)
            local = jnp.clip(p - base, 0, CHUNK - 1)
            plsc.store_scatter(chunk_v, [local], one16, mask=m)
        pltpu.sync_copy(chunk_v, ps_out.at[pl.ds(base, CHUNK)])

    @pl.when(jnp.logical_and(c == 0, s == 0))
    def _():
        pltpu.sync_copy(oseq_v, nseq_out)
        pltpu.sync_copy(onpu_v, nnpu_out)
        pltpu.sync_copy(ocp_v, ncp_out)
        pltpu.sync_copy(ocpp_v, ncpp_out)

    @pl.when(c == 1)
    def _():
        row4 = jnp.clip(s * PM_ROWS + lanes, 0, SLOTS - 1)
        m4 = lanes < PM_ROWS
        need4 = plsc.load_gather(nslot_v, [row4], mask=m4)
        page4 = plsc.load_gather(cslot_v, [row4], mask=m4)
        npu4 = plsc.load_gather(npub_v, [row4], mask=m4)
        wm = jnp.logical_and(m4, need4 == 1)
        plsc.store_scatter(
            pmbuf_v,
            [jnp.minimum(lanes, PM_ROWS - 1),
             jnp.clip(npu4, 0, MAX_PAGES_PER_SLOT - 1)],
            page4, mask=wm)
        pltpu.sync_copy(pmbuf_v, pm_out.at[pl.ds(s * PM_ROWS, PM_ROWS)])


_paged = pl.kernel(
    _body, out_type=_out_type, mesh=_mesh, scratch_types=_scratch,
    compiler_params=pltpu.CompilerParams(needs_layout_passes=False),
    name="page_manager_sc")


def kernel(page_status, page_map, sequence_lengths, num_pages_used,
           current_page, current_page_position):
    return _paged(page_status, page_map, sequence_lengths, num_pages_used,
                  current_page, current_page_position)


# speedup vs baseline: 8.8926x; 1.0003x over previous
import jax
import jax.numpy as jnp
from jax import lax
from jax.experimental import pallas as pl
from jax.experimental.pallas import tpu as pltpu
from jax.experimental.pallas import tpu_sc as plsc

NUM_PAGES = 32768
TOKENS_PER_PAGE = 64
SLOTS = 64
MAX_PAGES_PER_SLOT = 512

NC = 2
NS = 16
L = 16
CHUNK = NUM_PAGES // NS
NVREG = CHUNK // L
PM_ROWS = SLOTS // NS
SLOT_VREGS = SLOTS // L
SB_VREGS = 16
NSB = NVREG // SB_VREGS
PROW = SLOTS + L
SHROWS = 128

_mesh = plsc.VectorSubcoreMesh(
    core_axis_name="c", subcore_axis_name="s", num_cores=NC, num_subcores=NS)

_out_type = (
    jax.ShapeDtypeStruct((NUM_PAGES,), jnp.int32),
    jax.ShapeDtypeStruct((SLOTS, MAX_PAGES_PER_SLOT), jnp.int32),
    jax.ShapeDtypeStruct((SLOTS,), jnp.int32),
    jax.ShapeDtypeStruct((SLOTS,), jnp.int32),
    jax.ShapeDtypeStruct((SLOTS,), jnp.int32),
    jax.ShapeDtypeStruct((SLOTS,), jnp.int32),
)

_scratch = [
    pltpu.VMEM((CHUNK,), jnp.int32),
    pltpu.VMEM((PM_ROWS, MAX_PAGES_PER_SLOT), jnp.int32),
    pltpu.VMEM((PROW,), jnp.int32),
    pltpu.VMEM((NS, PROW), jnp.int32),
    pltpu.VMEM((L,), jnp.int32),
    pltpu.VMEM((L,), jnp.int32),
    pltpu.VMEM((SLOTS,), jnp.int32),
    pltpu.VMEM((SLOTS,), jnp.int32),
    pltpu.VMEM((SLOTS,), jnp.int32),
    pltpu.VMEM((SLOTS,), jnp.int32),
    pltpu.VMEM((SLOTS,), jnp.int32),
    pltpu.VMEM((SLOTS,), jnp.int32),
    pltpu.VMEM((SLOTS,), jnp.int32),
    pltpu.VMEM((SLOTS,), jnp.int32),
    pltpu.VMEM((SLOTS,), jnp.int32),
    pltpu.VMEM((SLOTS,), jnp.int32),
    pltpu.SMEM((1,), jnp.int32),
    pltpu.SemaphoreType.DMA,
    pltpu.VMEM_SHARED((SHROWS, PROW), jnp.int32),
]


def _body(ps_hbm, pm_hbm, seq_hbm, npu_hbm, cp_hbm, cpp_hbm,
          ps_out, pm_out, nseq_out, nnpu_out, ncp_out, ncpp_out,
          chunk_v, pmbuf_v, pos_v, posall_v, rexcl_v, rincl_v, claim_v,
          seqb_v, npub_v, cpb_v, cslot_v, nslot_v,
          oseq_v, onpu_v, ocp_v, ocpp_v,
          cnt_smem, dma_sem, shared_pos):
    del cpp_hbm
    c = lax.axis_index("c")
    s = lax.axis_index("s")
    lanes = lax.iota(jnp.int32, L)
    zero16 = jnp.zeros((L,), jnp.int32)
    one16 = jnp.full((L,), 1, jnp.int32)
    base = s * CHUNK

    cps = pltpu.async_copy(ps_hbm.at[pl.ds(base, CHUNK)], chunk_v, dma_sem)
    csq = pltpu.async_copy(seq_hbm, seqb_v, dma_sem)
    cnp = pltpu.async_copy(npu_hbm, npub_v, dma_sem)
    ccp = pltpu.async_copy(cp_hbm, cpb_v, dma_sem)
    cpm = pltpu.async_copy(
        pm_hbm.at[pl.ds(s * PM_ROWS, PM_ROWS)], pmbuf_v, dma_sem)
    cps.wait()
    csq.wait()
    cnp.wait()
    ccp.wait()
    cpm.wait()

    cnt_smem[0] = jnp.int32(0)

    def _sb(b, carry):
        cnt0 = cnt_smem[0]

        @pl.when(cnt0 < SLOTS)
        def _():
            cnt = cnt0
            for u in range(SB_VREGS):
                off = b * (SB_VREGS * L) + u * L
                v = chunk_v[pl.ds(off, L)]
                gidx = base + off + lanes
                m = jnp.logical_and(v == 0, gidx >= 1)
                pc = plsc.all_reduce_population_count(m)
                plsc.store_compressed(
                    pos_v.at[pl.ds(jnp.minimum(cnt, SLOTS), L)], gidx, mask=m)
                cnt = cnt + pc[0]
            cnt_smem[0] = cnt

        return carry

    lax.fori_loop(0, NSB, _sb, jnp.int32(0))
    cnt = cnt_smem[0]

    pos_v[pl.ds(SLOTS, L)] = jnp.broadcast_to(cnt, (L,))
    pltpu.sync_copy(pos_v, shared_pos.at[s])

    plsc.subcore_barrier()

    pltpu.sync_copy(shared_pos.at[pl.ds(0, NS)], posall_v)
    cnts16 = plsc.load_gather(posall_v, [lanes, jnp.full((L,), SLOTS, jnp.int32)])
    rincl = plsc.cumsum(cnts16)
    rexcl_v[...] = rincl - cnts16
    rincl_v[...] = rincl
    total16 = plsc.load_gather(rincl_v, [jnp.full((L,), NS - 1, jnp.int32)])

    for kg in range(SLOT_VREGS):
        rg = kg * L + lanes
        tsel = zero16
        rsel = zero16
        for t in range(NS):
            rt = plsc.load_gather(rexcl_v, [jnp.full((L,), t, jnp.int32)])
            cond = rt <= rg
            tsel = jnp.where(cond, jnp.full((L,), t, jnp.int32), tsel)
            rsel = jnp.where(cond, rt, rsel)
        local = jnp.clip(rg - rsel, 0, SLOTS - 1)
        validg = rg < total16
        pg = plsc.load_gather(posall_v, [tsel, local], mask=validg)
        claim_v[pl.ds(kg * L, L)] = jnp.where(validg, pg, 0)

    rank0 = jnp.int32(0)
    claims = []
    for k in range(SLOT_VREGS):
        off = k * L
        sq = seqb_v[pl.ds(off, L)]
        np_ = npub_v[pl.ds(off, L)]
        cp_ = cpb_v[pl.ds(off, L)]
        nsq = sq + jnp.where(cp_ == -1, 0, 1).astype(jnp.int32)
        nnp = (nsq + (TOKENS_PER_PAGE - 1)) // TOKENS_PER_PAGE
        ncpp = jnp.where(nsq == 0, 0, (nsq - 1) % TOKENS_PER_PAGE)
        needs = nnp > np_
        n32 = jnp.where(needs, 1, 0).astype(jnp.int32)
        csg = plsc.cumsum(n32)
        grank = rank0 + csg - 1
        rank0 = rank0 + jnp.sum(n32)
        rclip = jnp.clip(grank, 0, SLOTS - 1)
        page = plsc.load_gather(claim_v, [rclip], mask=needs)
        page = jnp.where(needs, page, 0)
        ncp = jnp.where(needs, page, cp_)
        oseq_v[pl.ds(off, L)] = nsq
        onpu_v[pl.ds(off, L)] = nnp
        ocp_v[pl.ds(off, L)] = ncp
        ocpp_v[pl.ds(off, L)] = ncpp
        cslot_v[pl.ds(off, L)] = page
        nslot_v[pl.ds(off, L)] = n32
        claims.append(jnp.where(needs, page, -1))

    @pl.when(c == 0)
    def _():
        for k in range(SLOT_VREGS):
            p = claims[k]
            m = jnp.logical_and(p >= base, p < base + CHUNK)
            local = jnp.clip(p - base, 0, CHUNK - 1)
            plsc.store_scatter(chunk_v, [local], one16, mask=m)
        pltpu.sync_copy(chunk_v, ps_out.at[pl.ds(base, CHUNK)])

    @pl.when(jnp.logical_and(c == 0, s == 0))
    def _():
        pltpu.sync_copy(oseq_v, nseq_out)
        pltpu.sync_copy(onpu_v, nnpu_out)
        pltpu.sync_copy(ocp_v, ncp_out)
        pltpu.sync_copy(ocpp_v, ncpp_out)

    @pl.when(c == 1)
    def _():
        row4 = jnp.clip(s * PM_ROWS + lanes, 0, SLOTS - 1)
        m4 = lanes < PM_ROWS
        need4 = plsc.load_gather(nslot_v, [row4], mask=m4)
        page4 = plsc.load_gather(cslot_v, [row4], mask=m4)
        npu4 = plsc.load_gather(npub_v, [row4], mask=m4)
        wm = jnp.logical_and(m4, need4 == 1)
        plsc.store_scatter(
            pmbuf_v,
            [jnp.minimum(lanes, PM_ROWS - 1),
             jnp.clip(npu4, 0, MAX_PAGES_PER_SLOT - 1)],
            page4, mask=wm)
        pltpu.sync_copy(pmbuf_v, pm_out.at[pl.ds(s * PM_ROWS, PM_ROWS)])

    plsc.subcore_barrier()


_paged = pl.kernel(
    _body, out_type=_out_type, mesh=_mesh, scratch_types=_scratch,
    compiler_params=pltpu.CompilerParams(needs_layout_passes=False),
    name="page_manager_sc")


def kernel(page_status, page_map, sequence_lengths, num_pages_used,
           current_page, current_page_position):
    return _paged(page_status, page_map, sequence_lengths, num_pages_used,
                  current_page, current_page_position)


# speedup vs baseline: 9.0532x; 1.0181x over previous
import jax
import jax.numpy as jnp
from jax import lax
from jax.experimental import pallas as pl
from jax.experimental.pallas import tpu as pltpu
from jax.experimental.pallas import tpu_sc as plsc

NUM_PAGES = 32768
TOKENS_PER_PAGE = 64
SLOTS = 64
MAX_PAGES_PER_SLOT = 512

NC = 2
NS = 16
L = 16
CHUNK = NUM_PAGES // NS
NVREG = CHUNK // L
PM_ROWS = SLOTS // NS
SLOT_VREGS = SLOTS // L
SB_VREGS = 16
NSB = NVREG // SB_VREGS
PROW = SLOTS + L
SHROWS = 128

_mesh = plsc.VectorSubcoreMesh(
    core_axis_name="c", subcore_axis_name="s", num_cores=NC, num_subcores=NS)

_out_type = (
    jax.ShapeDtypeStruct((NUM_PAGES,), jnp.int32),
    jax.ShapeDtypeStruct((SLOTS, MAX_PAGES_PER_SLOT), jnp.int32),
    jax.ShapeDtypeStruct((SLOTS,), jnp.int32),
    jax.ShapeDtypeStruct((SLOTS,), jnp.int32),
    jax.ShapeDtypeStruct((SLOTS,), jnp.int32),
    jax.ShapeDtypeStruct((SLOTS,), jnp.int32),
)

_scratch = [
    pltpu.VMEM((CHUNK,), jnp.int32),
    pltpu.VMEM((PM_ROWS, MAX_PAGES_PER_SLOT), jnp.int32),
    pltpu.VMEM((PROW,), jnp.int32),
    pltpu.VMEM((NS, PROW), jnp.int32),
    pltpu.VMEM((L,), jnp.int32),
    pltpu.VMEM((L,), jnp.int32),
    pltpu.VMEM((SLOTS,), jnp.int32),
    pltpu.VMEM((SLOTS,), jnp.int32),
    pltpu.VMEM((SLOTS,), jnp.int32),
    pltpu.VMEM((SLOTS,), jnp.int32),
    pltpu.VMEM((SLOTS,), jnp.int32),
    pltpu.VMEM((SLOTS,), jnp.int32),
    pltpu.VMEM((SLOTS,), jnp.int32),
    pltpu.VMEM((SLOTS,), jnp.int32),
    pltpu.VMEM((SLOTS,), jnp.int32),
    pltpu.VMEM((SLOTS,), jnp.int32),
    pltpu.SMEM((1,), jnp.int32),
    pltpu.SemaphoreType.DMA,
    pltpu.VMEM_SHARED((SHROWS, PROW), jnp.int32),
]


def _body(ps_hbm, pm_hbm, seq_hbm, npu_hbm, cp_hbm, cpp_hbm,
          ps_out, pm_out, nseq_out, nnpu_out, ncp_out, ncpp_out,
          chunk_v, pmbuf_v, pos_v, posall_v, rexcl_v, rincl_v, claim_v,
          seqb_v, npub_v, cpb_v, cslot_v, nslot_v,
          oseq_v, onpu_v, ocp_v, ocpp_v,
          cnt_smem, dma_sem, shared_pos):
    del cpp_hbm
    c = lax.axis_index("c")
    s = lax.axis_index("s")
    lanes = lax.iota(jnp.int32, L)
    zero16 = jnp.zeros((L,), jnp.int32)
    one16 = jnp.full((L,), 1, jnp.int32)
    base = s * CHUNK

    cps = pltpu.async_copy(ps_hbm.at[pl.ds(base, CHUNK)], chunk_v, dma_sem)
    csq = pltpu.async_copy(seq_hbm, seqb_v, dma_sem)
    cnp = pltpu.async_copy(npu_hbm, npub_v, dma_sem)
    ccp = pltpu.async_copy(cp_hbm, cpb_v, dma_sem)
    cpm = pltpu.async_copy(
        pm_hbm.at[pl.ds(s * PM_ROWS, PM_ROWS)], pmbuf_v, dma_sem)
    cps.wait()
    csq.wait()
    cnp.wait()
    ccp.wait()
    cpm.wait()

    anyneeds = jnp.zeros((L,), jnp.bool_)
    for k in range(SLOT_VREGS):
        off = k * L
        sq = seqb_v[pl.ds(off, L)]
        np_ = npub_v[pl.ds(off, L)]
        cp_ = cpb_v[pl.ds(off, L)]
        nsq = sq + jnp.where(cp_ == -1, 0, 1).astype(jnp.int32)
        nnp = (nsq + (TOKENS_PER_PAGE - 1)) // TOKENS_PER_PAGE
        anyneeds = jnp.logical_or(anyneeds, nnp > np_)
    kpc = plsc.all_reduce_population_count(anyneeds)
    has_work = kpc[0] > 0

    cnt_smem[0] = jnp.int32(0)

    def _sb(b, carry):
        cnt0 = cnt_smem[0]

        @pl.when(cnt0 < SLOTS)
        def _():
            cnt = cnt0
            for u in range(SB_VREGS):
                off = b * (SB_VREGS * L) + u * L
                v = chunk_v[pl.ds(off, L)]
                gidx = base + off + lanes
                m = jnp.logical_and(v == 0, gidx >= 1)
                pc = plsc.all_reduce_population_count(m)
                plsc.store_compressed(
                    pos_v.at[pl.ds(jnp.minimum(cnt, SLOTS), L)], gidx, mask=m)
                cnt = cnt + pc[0]
            cnt_smem[0] = cnt

        return carry

    @pl.when(has_work)
    def _():
        lax.fori_loop(0, NSB, _sb, jnp.int32(0))
        cnt = cnt_smem[0]

        pos_v[pl.ds(SLOTS, L)] = jnp.broadcast_to(cnt, (L,))
        pltpu.sync_copy(pos_v, shared_pos.at[s])

        plsc.subcore_barrier()

        pltpu.sync_copy(shared_pos.at[pl.ds(0, NS)], posall_v)
        cnts16 = plsc.load_gather(
            posall_v, [lanes, jnp.full((L,), SLOTS, jnp.int32)])
        rincl = plsc.cumsum(cnts16)
        rexcl_v[...] = rincl - cnts16
        rincl_v[...] = rincl
        total16 = plsc.load_gather(rincl_v, [jnp.full((L,), NS - 1, jnp.int32)])

        for kg in range(SLOT_VREGS):
            rg = kg * L + lanes
            tsel = zero16
            rsel = zero16
            for t in range(NS):
                rt = plsc.load_gather(rexcl_v, [jnp.full((L,), t, jnp.int32)])
                cond = rt <= rg
                tsel = jnp.where(cond, jnp.full((L,), t, jnp.int32), tsel)
                rsel = jnp.where(cond, rt, rsel)
            local = jnp.clip(rg - rsel, 0, SLOTS - 1)
            validg = rg < total16
            pg = plsc.load_gather(posall_v, [tsel, local], mask=validg)
            claim_v[pl.ds(kg * L, L)] = jnp.where(validg, pg, 0)

        plsc.subcore_barrier()

    rank0 = jnp.int32(0)
    claims = []
    for k in range(SLOT_VREGS):
        off = k * L
        sq = seqb_v[pl.ds(off, L)]
        np_ = npub_v[pl.ds(off, L)]
        cp_ = cpb_v[pl.ds(off, L)]
        nsq = sq + jnp.where(cp_ == -1, 0, 1).astype(jnp.int32)
        nnp = (nsq + (TOKENS_PER_PAGE - 1)) // TOKENS_PER_PAGE
        ncpp = jnp.where(nsq == 0, 0, (nsq - 1) % TOKENS_PER_PAGE)
        needs = nnp > np_
        n32 = jnp.where(needs, 1, 0).astype(jnp.int32)
        csg = plsc.cumsum(n32)
        grank = rank0 + csg - 1
        rank0 = rank0 + jnp.sum(n32)
        rclip = jnp.clip(grank, 0, SLOTS - 1)
        page = plsc.load_gather(claim_v, [rclip], mask=needs)
        page = jnp.where(needs, page, 0)
        ncp = jnp.where(needs, page, cp_)
        oseq_v[pl.ds(off, L)] = nsq
        onpu_v[pl.ds(off, L)] = nnp
        ocp_v[pl.ds(off, L)] = ncp
        ocpp_v[pl.ds(off, L)] = ncpp
        cslot_v[pl.ds(off, L)] = page
        nslot_v[pl.ds(off, L)] = n32
        claims.append(jnp.where(needs, page, -1))

    @pl.when(c == 0)
    def _():
        for k in range(SLOT_VREGS):
            p = claims[k]
            m = jnp.logical_and(p >= base, p < base + CHUNK)
            local = jnp.clip(p - base, 0, CHUNK - 1)
            plsc.store_scatter(chunk_v, [local], one16, mask=m)
        pltpu.sync_copy(chunk_v, ps_out.at[pl.ds(base, CHUNK)])

    @pl.when(jnp.logical_and(c == 0, s == 0))
    def _():
        pltpu.sync_copy(oseq_v, nseq_out)
        pltpu.sync_copy(onpu_v, nnpu_out)
        pltpu.sync_copy(ocp_v, ncp_out)
        pltpu.sync_copy(ocpp_v, ncpp_out)

    @pl.when(c == 1)
    def _():
        row4 = jnp.clip(s * PM_ROWS + lanes, 0, SLOTS - 1)
        m4 = lanes < PM_ROWS
        need4 = plsc.load_gather(nslot_v, [row4], mask=m4)
        page4 = plsc.load_gather(cslot_v, [row4], mask=m4)
        npu4 = plsc.load_gather(npub_v, [row4], mask=m4)
        wm = jnp.logical_and(m4, need4 == 1)
        plsc.store_scatter(
            pmbuf_v,
            [jnp.minimum(lanes, PM_ROWS - 1),
             jnp.clip(npu4, 0, MAX_PAGES_PER_SLOT - 1)],
            page4, mask=wm)
        pltpu.sync_copy(pmbuf_v, pm_out.at[pl.ds(s * PM_ROWS, PM_ROWS)])


_paged = pl.kernel(
    _body, out_type=_out_type, mesh=_mesh, scratch_types=_scratch,
    compiler_params=pltpu.CompilerParams(needs_layout_passes=False),
    name="page_manager_sc")


def kernel(page_status, page_map, sequence_lengths, num_pages_used,
           current_page, current_page_position):
    return _paged(page_status, page_map, sequence_lengths, num_pages_used,
                  current_page, current_page_position)


# speedup vs baseline: 9.1036x; 1.0056x over previous
import jax
import jax.numpy as jnp
from jax import lax
from jax.experimental import pallas as pl
from jax.experimental.pallas import tpu as pltpu
from jax.experimental.pallas import tpu_sc as plsc

NUM_PAGES = 32768
TOKENS_PER_PAGE = 64
SLOTS = 64
MAX_PAGES_PER_SLOT = 512

NC = 2
NS = 16
L = 16
CHUNK = NUM_PAGES // NS
NVREG = CHUNK // L
PM_ROWS = SLOTS // NS
SLOT_VREGS = SLOTS // L
SB_VREGS = 16
NSB = NVREG // SB_VREGS
PROW = SLOTS + L
SHROWS = 128

_mesh = plsc.VectorSubcoreMesh(
    core_axis_name="c", subcore_axis_name="s", num_cores=NC, num_subcores=NS)

_out_type = (
    jax.ShapeDtypeStruct((NUM_PAGES,), jnp.int32),
    jax.ShapeDtypeStruct((SLOTS, MAX_PAGES_PER_SLOT), jnp.int32),
    jax.ShapeDtypeStruct((SLOTS,), jnp.int32),
    jax.ShapeDtypeStruct((SLOTS,), jnp.int32),
    jax.ShapeDtypeStruct((SLOTS,), jnp.int32),
    jax.ShapeDtypeStruct((SLOTS,), jnp.int32),
)

_scratch = [
    pltpu.VMEM((CHUNK,), jnp.int32),
    pltpu.VMEM((PM_ROWS, MAX_PAGES_PER_SLOT), jnp.int32),
    pltpu.VMEM((PROW,), jnp.int32),
    pltpu.VMEM((NS, PROW), jnp.int32),
    pltpu.VMEM((L,), jnp.int32),
    pltpu.VMEM((L,), jnp.int32),
    pltpu.VMEM((SLOTS,), jnp.int32),
    pltpu.VMEM((SLOTS,), jnp.int32),
    pltpu.VMEM((SLOTS,), jnp.int32),
    pltpu.VMEM((SLOTS,), jnp.int32),
    pltpu.VMEM((SLOTS,), jnp.int32),
    pltpu.VMEM((SLOTS,), jnp.int32),
    pltpu.VMEM((SLOTS,), jnp.int32),
    pltpu.VMEM((SLOTS,), jnp.int32),
    pltpu.VMEM((SLOTS,), jnp.int32),
    pltpu.VMEM((SLOTS,), jnp.int32),
    pltpu.SMEM((1,), jnp.int32),
    pltpu.SemaphoreType.DMA,
    pltpu.VMEM_SHARED((SHROWS, PROW), jnp.int32),
]


def _body(ps_hbm, pm_hbm, seq_hbm, npu_hbm, cp_hbm, cpp_hbm,
          ps_out, pm_out, nseq_out, nnpu_out, ncp_out, ncpp_out,
          chunk_v, pmbuf_v, pos_v, posall_v, rexcl_v, rincl_v, claim_v,
          seqb_v, npub_v, cpb_v, cslot_v, nslot_v,
          oseq_v, onpu_v, ocp_v, ocpp_v,
          cnt_smem, dma_sem, shared_pos):
    del cpp_hbm
    c = lax.axis_index("c")
    s = lax.axis_index("s")
    lanes = lax.iota(jnp.int32, L)
    zero16 = jnp.zeros((L,), jnp.int32)
    one16 = jnp.full((L,), 1, jnp.int32)
    base = s * CHUNK

    cps = pltpu.async_copy(ps_hbm.at[pl.ds(base, CHUNK)], chunk_v, dma_sem)
    csq = pltpu.async_copy(seq_hbm, seqb_v, dma_sem)
    cnp = pltpu.async_copy(npu_hbm, npub_v, dma_sem)
    ccp = pltpu.async_copy(cp_hbm, cpb_v, dma_sem)
    cpm = pltpu.async_copy(
        pm_hbm.at[pl.ds(s * PM_ROWS, PM_ROWS)], pmbuf_v, dma_sem)
    cps.wait()
    csq.wait()
    cnp.wait()
    ccp.wait()
    cpm.wait()

    anyneeds = jnp.zeros((L,), jnp.bool_)
    for k in range(SLOT_VREGS):
        off = k * L
        sq = seqb_v[pl.ds(off, L)]
        np_ = npub_v[pl.ds(off, L)]
        cp_ = cpb_v[pl.ds(off, L)]
        nsq = sq + jnp.where(cp_ == -1, 0, 1).astype(jnp.int32)
        nnp = (nsq + (TOKENS_PER_PAGE - 1)) // TOKENS_PER_PAGE
        anyneeds = jnp.logical_or(anyneeds, nnp > np_)
    kpc = plsc.all_reduce_population_count(anyneeds)
    has_work = kpc[0] > 0

    cnt_smem[0] = jnp.int32(0)

    def _sb(b, carry):
        cnt0 = cnt_smem[0]

        @pl.when(cnt0 < SLOTS)
        def _():
            cnt = cnt0
            for u in range(SB_VREGS):
                off = b * (SB_VREGS * L) + u * L
                v = chunk_v[pl.ds(off, L)]
                gidx = base + off + lanes
                m = jnp.logical_and(v == 0, gidx >= 1)
                pc = plsc.all_reduce_population_count(m)
                plsc.store_compressed(
                    pos_v.at[pl.ds(jnp.minimum(cnt, SLOTS), L)], gidx, mask=m)
                cnt = cnt + pc[0]
            cnt_smem[0] = cnt

        return carry

    @pl.when(has_work)
    def _():
        lax.fori_loop(0, NSB, _sb, jnp.int32(0))
        cnt = cnt_smem[0]

        pos_v[pl.ds(SLOTS, L)] = jnp.broadcast_to(cnt, (L,))
        pltpu.sync_copy(pos_v, shared_pos.at[s])

        plsc.subcore_barrier()

        pltpu.sync_copy(shared_pos.at[pl.ds(0, NS)], posall_v)
        cnts16 = plsc.load_gather(
            posall_v, [lanes, jnp.full((L,), SLOTS, jnp.int32)])
        rincl = plsc.cumsum(cnts16)
        rexcl_v[...] = rincl - cnts16
        rincl_v[...] = rincl
        total16 = plsc.load_gather(rincl_v, [jnp.full((L,), NS - 1, jnp.int32)])

        for kg in range(SLOT_VREGS):
            rg = kg * L + lanes
            tsel = zero16
            rsel = zero16
            for t in range(NS):
                rt = plsc.load_gather(rexcl_v, [jnp.full((L,), t, jnp.int32)])
                cond = rt <= rg
                tsel = jnp.where(cond, jnp.full((L,), t, jnp.int32), tsel)
                rsel = jnp.where(cond, rt, rsel)
            local = jnp.clip(rg - rsel, 0, SLOTS - 1)
            validg = rg < total16
            pg = plsc.load_gather(posall_v, [tsel, local], mask=validg)
            claim_v[pl.ds(kg * L, L)] = jnp.where(validg, pg, 0)

        plsc.subcore_barrier()

    rank0 = jnp.int32(0)
    claims = []
    for k in range(SLOT_VREGS):
        off = k * L
        sq = seqb_v[pl.ds(off, L)]
        np_ = npub_v[pl.ds(off, L)]
        cp_ = cpb_v[pl.ds(off, L)]
        nsq = sq + jnp.where(cp_ == -1, 0, 1).astype(jnp.int32)
        nnp = (nsq + (TOKENS_PER_PAGE - 1)) // TOKENS_PER_PAGE
        ncpp = jnp.where(nsq == 0, 0, (nsq - 1) % TOKENS_PER_PAGE)
        needs = nnp > np_
        n32 = jnp.where(needs, 1, 0).astype(jnp.int32)
        csg = plsc.cumsum(n32)
        grank = rank0 + csg - 1
        rank0 = rank0 + jnp.sum(n32)
        rclip = jnp.clip(grank, 0, SLOTS - 1)
        page = plsc.load_gather(claim_v, [rclip], mask=needs)
        page = jnp.where(needs, page, 0)
        ncp = jnp.where(needs, page, cp_)
        oseq_v[pl.ds(off, L)] = nsq
        onpu_v[pl.ds(off, L)] = nnp
        ocp_v[pl.ds(off, L)] = ncp
        ocpp_v[pl.ds(off, L)] = ncpp
        cslot_v[pl.ds(off, L)] = page
        nslot_v[pl.ds(off, L)] = n32
        claims.append(jnp.where(needs, page, -1))

    @pl.when(c == 0)
    def _():
        for k in range(SLOT_VREGS):
            p = claims[k]
            m = jnp.logical_and(p >= base, p < base + CHUNK)
            local = jnp.clip(p - base, 0, CHUNK - 1)
            plsc.store_scatter(chunk_v, [local], one16, mask=m)
        pltpu.sync_copy(chunk_v, ps_out.at[pl.ds(base, CHUNK)])

    @pl.when(jnp.logical_and(c == 0, s == 1))
    def _():
        pltpu.sync_copy(oseq_v, nseq_out)

    @pl.when(jnp.logical_and(c == 0, s == 2))
    def _():
        pltpu.sync_copy(onpu_v, nnpu_out)

    @pl.when(jnp.logical_and(c == 1, s == 1))
    def _():
        pltpu.sync_copy(ocp_v, ncp_out)

    @pl.when(jnp.logical_and(c == 1, s == 2))
    def _():
        pltpu.sync_copy(ocpp_v, ncpp_out)

    @pl.when(c == 1)
    def _():
        row4 = jnp.clip(s * PM_ROWS + lanes, 0, SLOTS - 1)
        m4 = lanes < PM_ROWS
        need4 = plsc.load_gather(nslot_v, [row4], mask=m4)
        page4 = plsc.load_gather(cslot_v, [row4], mask=m4)
        npu4 = plsc.load_gather(npub_v, [row4], mask=m4)
        wm = jnp.logical_and(m4, need4 == 1)
        plsc.store_scatter(
            pmbuf_v,
            [jnp.minimum(lanes, PM_ROWS - 1),
             jnp.clip(npu4, 0, MAX_PAGES_PER_SLOT - 1)],
            page4, mask=wm)
        pltpu.sync_copy(pmbuf_v, pm_out.at[pl.ds(s * PM_ROWS, PM_ROWS)])


_paged = pl.kernel(
    _body, out_type=_out_type, mesh=_mesh, scratch_types=_scratch,
    compiler_params=pltpu.CompilerParams(needs_layout_passes=False),
    name="page_manager_sc")


def kernel(page_status, page_map, sequence_lengths, num_pages_used,
           current_page, current_page_position):
    return _paged(page_status, page_map, sequence_lengths, num_pages_used,
                  current_page, current_page_position)
